# Initial kernel scaffold; baseline (speedup 1.0000x reference)
import jax
import jax.numpy as jnp
from jax.experimental import pallas as pl


def kernel(x1_node_features, x1_edge_features, x1_from_idx, x1_to_idx, x1_graph_idx, x2_node_features, x2_edge_features, x2_from_idx, x2_to_idx, x2_graph_idx, n_graphs, W_nenc, b_nenc, W_eenc, b_eenc, W_msg1, b_msg1, W_msg2, b_msg2, W_upd1, b_upd1, W_upd2, b_upd2, W_gate, b_gate, W_gout, b_gout):
    raise NotImplementedError("write your pallas kernel here")


# speedup vs baseline: 3.0930x; 3.0930x over previous
import functools

import jax
import jax.numpy as jnp
from jax import lax
from jax.experimental import pallas as pl
from jax.experimental.pallas import tpu as pltpu
from jax.experimental.pallas import tpu_sc as plsc

N = 10000
E = 320000
D = 128
DE = 16
G = 128
NG = 128
NPROP = 5
T = 2

BN = 1000
NB = N // BN
EBLK = 4000
NEB = E // EBLK

NSUB = 16
SB = 80
EPT = E // NSUB
NBLK = EPT // SB
CH = 80
NCH = N // CH

F32 = jnp.float32


def _dot(a, b):
    return jnp.dot(a, b, preferred_element_type=F32)


def _k_pre_body(nf, wn, bn, w1s, w1d, h, hs, hd):
    x = _dot(nf[0], wn[...]) + bn[...]
    h[0] = x
    hs[0] = _dot(x, w1s[...])
    hd[0] = _dot(x, w1d[...])


def _k_ee_body(ef, we, be, w1e, bm1, ee):
    t = _dot(ef[0], we[...]) + be[...]
    ee[0] = _dot(t, w1e[...]) + bm1[...]


def _update(h, s, cnt, wm2, bm2, wuh, wua, bu1, wu2, bu2):
    agg = _dot(s[0], wm2[...]) + cnt[0][:, 0:1] * bm2[...]
    u = _dot(h[0], wuh[...]) + _dot(agg, wua[...]) + bu1[...]
    return h[0] + _dot(jnp.maximum(u, 0.0), wu2[...]) + bu2[...]


def _k_mid_body(h, s, cnt, wm2, bm2, wuh, wua, bu1, wu2, bu2, w1s, w1d,
                hn, hs, hd):
    x = _update(h, s, cnt, wm2, bm2, wuh, wua, bu1, wu2, bu2)
    hn[0] = x
    hs[0] = _dot(x, w1s[...])
    hd[0] = _dot(x, w1d[...])


def _k_post_body(h, s, cnt, wm2, bm2, wuh, wua, bu1, wu2, bu2,
                 wg, bg, gi, wgo, bgo, out, acc):
    i = pl.program_id(1)
    x = _update(h, s, cnt, wm2, bm2, wuh, wua, bu1, wu2, bu2)
    gv = _dot(x, wg[...]) + bg[...]
    gated = jax.nn.sigmoid(gv[:, :G]) * gv[:, G:]
    seg = gi[0, 0]
    onehot = (seg[:, None] ==
              lax.broadcasted_iota(jnp.int32, (BN, NG), 1)).astype(F32)
    contrib = lax.dot_general(onehot, gated, (((0,), (0,)), ((), ())),
                              preferred_element_type=F32)

    @pl.when(i == 0)
    def _():
        acc[...] = contrib

    @pl.when(i > 0)
    def _():
        acc[...] = acc[...] + contrib

    @pl.when(i == NB - 1)
    def _():
        out[0] = _dot(acc[...], wgo[...]) + bgo[...]


def _nblock(last):
    return pl.BlockSpec((1, BN, last), lambda t, i: (t, i, 0))


def _wspec(shape):
    return pl.BlockSpec(shape, lambda t, i: tuple(0 for _ in shape))


_k_pre = pl.pallas_call(
    _k_pre_body,
    grid=(T, NB),
    in_specs=[_nblock(D), _wspec((D, D)), _wspec((1, D)),
              _wspec((D, D)), _wspec((D, D))],
    out_specs=[_nblock(D), _nblock(D), _nblock(D)],
    out_shape=[jax.ShapeDtypeStruct((T, N, D), F32)] * 3,
)

_k_ee = pl.pallas_call(
    _k_ee_body,
    grid=(T, NEB),
    in_specs=[pl.BlockSpec((1, EBLK, DE), lambda t, i: (t, i, 0)),
              _wspec((DE, DE)), _wspec((1, DE)),
              _wspec((DE, D)), _wspec((1, D))],
    out_specs=pl.BlockSpec((1, EBLK, D), lambda t, i: (t, i, 0)),
    out_shape=jax.ShapeDtypeStruct((T, E, D), F32),
)

_k_mid = pl.pallas_call(
    _k_mid_body,
    grid=(T, NB),
    in_specs=[_nblock(D), _nblock(D), _nblock(16),
              _wspec((D, D)), _wspec((1, D)),
              _wspec((D, D)), _wspec((D, D)), _wspec((1, D)),
              _wspec((D, D)), _wspec((1, D)),
              _wspec((D, D)), _wspec((D, D))],
    out_specs=[_nblock(D), _nblock(D), _nblock(D)],
    out_shape=[jax.ShapeDtypeStruct((T, N, D), F32)] * 3,
)

_k_post = pl.pallas_call(
    _k_post_body,
    grid=(T, NB),
    in_specs=[_nblock(D), _nblock(D), _nblock(16),
              _wspec((D, D)), _wspec((1, D)),
              _wspec((D, D)), _wspec((D, D)), _wspec((1, D)),
              _wspec((D, D)), _wspec((1, D)),
              _wspec((D, 2 * G)), _wspec((1, 2 * G)),
              pl.BlockSpec((1, 1, BN), lambda t, i: (t * NB + i, 0, 0)),
              _wspec((G, G)), _wspec((1, G))],
    out_specs=pl.BlockSpec((1, NG, G), lambda t, i: (t, 0, 0)),
    out_shape=jax.ShapeDtypeStruct((T, NG, G), F32),
    scratch_shapes=[pltpu.VMEM((NG, G), F32)],
)


_SC_MESH = plsc.VectorSubcoreMesh(core_axis_name="c", subcore_axis_name="s")


@functools.partial(
    pl.kernel,
    out_type=jax.ShapeDtypeStruct((T * N, D), F32),
    mesh=_SC_MESH,
    scratch_types=[
        pltpu.VMEM_SHARED((N, D), F32),
        pltpu.VMEM((SB,), jnp.int32),
        pltpu.VMEM((SB,), jnp.int32),
        pltpu.VMEM((SB,), jnp.int32),
        pltpu.VMEM((SB, D), F32),
        pltpu.VMEM((SB, D), F32),
        pltpu.VMEM((SB, D), F32),
        pltpu.VMEM((CH, D), F32),
        pltpu.SemaphoreType.DMA,
        pltpu.SemaphoreType.DMA,
    ],
)
def _k_edge(hs_hbm, hd_hbm, ee_hbm, fi_hbm, ti_hbm, out_hbm,
            acc, fib, tib, tibg, abuf, bbuf, cbuf, zbuf, sem_a, sem_b):
    c = lax.axis_index("c")
    s = lax.axis_index("s")
    ch0 = 7 * s + jnp.minimum(s, 13)
    nch = 7 + (s < 13).astype(jnp.int32)

    def zrow(r, carry):
        for k in range(D // 16):
            zbuf[r, pl.ds(k * 16, 16)] = jnp.zeros((16,), F32)
        return carry

    lax.fori_loop(0, CH, zrow, 0)

    def zchunk(j, carry):
        pltpu.sync_copy(zbuf, acc.at[pl.ds((ch0 + j) * CH, CH)])
        return carry

    lax.fori_loop(0, nch, zchunk, 0)
    plsc.subcore_barrier()

    coff = c * N

    def blk_body(blk, carry):
        e0 = c * E + s * EPT + blk * SB
        pltpu.sync_copy(fi_hbm.at[pl.ds(e0, SB)], fib)
        pltpu.sync_copy(ti_hbm.at[pl.ds(e0, SB)], tib)
        for k in range(SB // 16):
            sl = pl.ds(k * 16, 16)
            fib[sl] = fib[sl] + coff
            tibg[sl] = tib[sl] + coff
        cp_a = pltpu.async_copy(hs_hbm.at[fib], abuf, sem_a)
        cp_b = pltpu.async_copy(hd_hbm.at[tibg], bbuf, sem_b)
        pltpu.sync_copy(ee_hbm.at[pl.ds(e0, SB)], cbuf)
        cp_a.wait()
        cp_b.wait()

        def row(r, rc):
            for k in range(D // 16):
                sl = pl.ds(k * 16, 16)
                v = abuf[r, sl] + bbuf[r, sl] + cbuf[r, sl]
                cbuf[r, sl] = jnp.maximum(v, 0.0)
            return rc

        lax.fori_loop(0, SB, row, 0)
        pltpu.sync_copy(cbuf, acc.at[tib], add=True)
        return carry

    lax.fori_loop(0, NBLK, blk_body, 0)
    plsc.subcore_barrier()

    def outchunk(j, carry):
        pltpu.sync_copy(acc.at[pl.ds((ch0 + j) * CH, CH)],
                        out_hbm.at[pl.ds(c * N + (ch0 + j) * CH, CH)])
        return carry

    lax.fori_loop(0, nch, outchunk, 0)


@functools.partial(
    pl.kernel,
    out_type=jax.ShapeDtypeStruct((T * N, 16), F32),
    mesh=_SC_MESH,
    scratch_types=[
        pltpu.VMEM_SHARED((N, 16), F32),
        pltpu.VMEM((SB,), jnp.int32),
        pltpu.VMEM((SB, 16), F32),
        pltpu.VMEM((CH, 16), F32),
    ],
)
def _k_cnt(ti_hbm, out_hbm, acc, tib, ones, zbuf):
    c = lax.axis_index("c")
    s = lax.axis_index("s")
    ch0 = 7 * s + jnp.minimum(s, 13)
    nch = 7 + (s < 13).astype(jnp.int32)

    def fill(r, carry):
        ones[r, pl.ds(0, 16)] = jnp.full((16,), 1.0, F32)
        return carry

    lax.fori_loop(0, SB, fill, 0)

    def zrow(r, carry):
        zbuf[r, pl.ds(0, 16)] = jnp.zeros((16,), F32)
        return carry

    lax.fori_loop(0, CH, zrow, 0)

    def zchunk(j, carry):
        pltpu.sync_copy(zbuf, acc.at[pl.ds((ch0 + j) * CH, CH)])
        return carry

    lax.fori_loop(0, nch, zchunk, 0)
    plsc.subcore_barrier()

    def blk_body(blk, carry):
        e0 = c * E + s * EPT + blk * SB
        pltpu.sync_copy(ti_hbm.at[pl.ds(e0, SB)], tib)
        pltpu.sync_copy(ones, acc.at[tib], add=True)
        return carry

    lax.fori_loop(0, NBLK, blk_body, 0)
    plsc.subcore_barrier()

    def outchunk(j, carry):
        pltpu.sync_copy(acc.at[pl.ds((ch0 + j) * CH, CH)],
                        out_hbm.at[pl.ds(c * N + (ch0 + j) * CH, CH)])
        return carry

    lax.fori_loop(0, nch, outchunk, 0)


def kernel(x1_node_features, x1_edge_features, x1_from_idx, x1_to_idx,
           x1_graph_idx, x2_node_features, x2_edge_features, x2_from_idx,
           x2_to_idx, x2_graph_idx, n_graphs,
           W_nenc, b_nenc, W_eenc, b_eenc,
           W_msg1, b_msg1, W_msg2, b_msg2,
           W_upd1, b_upd1, W_upd2, b_upd2,
           W_gate, b_gate, W_gout, b_gout):
    del n_graphs
    nf = jnp.stack([x1_node_features, x2_node_features])
    ef = jnp.stack([x1_edge_features, x2_edge_features])
    fi2 = jnp.concatenate([x1_from_idx, x2_from_idx])
    ti2 = jnp.concatenate([x1_to_idx, x2_to_idx])
    gi3 = jnp.stack([x1_graph_idx, x2_graph_idx]).reshape(T * NB, 1, BN)

    W1s, W1d, W1e = W_msg1[:D], W_msg1[D:2 * D], W_msg1[2 * D:]
    Wuh, Wua = W_upd1[:D], W_upd1[D:]
    r1 = lambda b: b.reshape(1, -1)

    h, hs, hd = _k_pre(nf, W_nenc, r1(b_nenc), W1s, W1d)
    ee = _k_ee(ef, W_eenc, r1(b_eenc), W1e, r1(b_msg1))
    cnt = _k_cnt(ti2).reshape(T, N, 16)
    ee2 = ee.reshape(T * E, D)

    upd_args = (W_msg2, r1(b_msg2), Wuh, Wua, r1(b_upd1),
                W_upd2, r1(b_upd2))
    s = None
    for layer in range(NPROP):
        s = _k_edge(hs.reshape(T * N, D), hd.reshape(T * N, D),
                    ee2, fi2, ti2).reshape(T, N, D)
        if layer < NPROP - 1:
            h, hs, hd = _k_mid(h, s, cnt, *upd_args, W1s, W1d)

    out = _k_post(h, s, cnt, *upd_args,
                  W_gate, r1(b_gate), gi3, W_gout, r1(b_gout))
    return out[0], out[1]


# speedup vs baseline: 4.3805x; 1.4163x over previous
import functools

import jax
import jax.numpy as jnp
from jax import lax
from jax.experimental import pallas as pl
from jax.experimental.pallas import tpu as pltpu
from jax.experimental.pallas import tpu_sc as plsc

N = 10000
E = 320000
D = 128
DE = 16
G = 128
NG = 128
NPROP = 5
T = 2

BN = 1000
NB = N // BN
EBLK = 4000
NEB = E // EBLK

NSUB = 16
SB = 80
EPT = E // NSUB
NBLK = EPT // SB
CH = 80
NCH = N // CH
SBK = 64
GB = E // SBK
NMAIN = 312
NPAIR = NMAIN // 2
CHZ = 40

F32 = jnp.float32


def _dot(a, b):
    return jnp.dot(a, b, preferred_element_type=F32)


def _k_pre_body(nf, wn, bn, w1s, w1d, h, hs, hd):
    x = _dot(nf[0], wn[...]) + bn[...]
    h[0] = x
    hs[0] = _dot(x, w1s[...])
    hd[0] = _dot(x, w1d[...])


def _k_ee_body(ef, we, be, w1e, bm1, ee):
    t = _dot(ef[0], we[...]) + be[...]
    ee[0] = _dot(t, w1e[...]) + bm1[...]


def _update(h, s, cnt, wm2, bm2, wuh, wua, bu1, wu2, bu2):
    agg = _dot(s[0], wm2[...]) + cnt[0][:, 0:1] * bm2[...]
    u = _dot(h[0], wuh[...]) + _dot(agg, wua[...]) + bu1[...]
    return h[0] + _dot(jnp.maximum(u, 0.0), wu2[...]) + bu2[...]


def _k_mid_body(h, s, cnt, wm2, bm2, wuh, wua, bu1, wu2, bu2, w1s, w1d,
                hn, hs, hd):
    x = _update(h, s, cnt, wm2, bm2, wuh, wua, bu1, wu2, bu2)
    hn[0] = x
    hs[0] = _dot(x, w1s[...])
    hd[0] = _dot(x, w1d[...])


def _k_post_body(h, s, cnt, wm2, bm2, wuh, wua, bu1, wu2, bu2,
                 wg, bg, gi, wgo, bgo, out, acc):
    i = pl.program_id(1)
    x = _update(h, s, cnt, wm2, bm2, wuh, wua, bu1, wu2, bu2)
    gv = _dot(x, wg[...]) + bg[...]
    gated = jax.nn.sigmoid(gv[:, :G]) * gv[:, G:]
    seg = gi[0, 0]
    onehot = (seg[:, None] ==
              lax.broadcasted_iota(jnp.int32, (BN, NG), 1)).astype(F32)
    contrib = lax.dot_general(onehot, gated, (((0,), (0,)), ((), ())),
                              preferred_element_type=F32)

    @pl.when(i == 0)
    def _():
        acc[...] = contrib

    @pl.when(i > 0)
    def _():
        acc[...] = acc[...] + contrib

    @pl.when(i == NB - 1)
    def _():
        out[0] = _dot(acc[...], wgo[...]) + bgo[...]


def _nblock(last):
    return pl.BlockSpec((1, BN, last), lambda t, i: (t, i, 0))


def _wspec(shape):
    return pl.BlockSpec(shape, lambda t, i: tuple(0 for _ in shape))


_k_pre = pl.pallas_call(
    _k_pre_body,
    grid=(T, NB),
    in_specs=[_nblock(D), _wspec((D, D)), _wspec((1, D)),
              _wspec((D, D)), _wspec((D, D))],
    out_specs=[_nblock(D), _nblock(D), _nblock(D)],
    out_shape=[jax.ShapeDtypeStruct((T, N, D), F32)] * 3,
)

_k_ee = pl.pallas_call(
    _k_ee_body,
    grid=(T, NEB),
    in_specs=[pl.BlockSpec((1, EBLK, DE), lambda t, i: (t, i, 0)),
              _wspec((DE, DE)), _wspec((1, DE)),
              _wspec((DE, D)), _wspec((1, D))],
    out_specs=pl.BlockSpec((1, EBLK, D), lambda t, i: (t, i, 0)),
    out_shape=jax.ShapeDtypeStruct((T, E, D), F32),
)

_k_mid = pl.pallas_call(
    _k_mid_body,
    grid=(T, NB),
    in_specs=[_nblock(D), _nblock(D), _nblock(16),
              _wspec((D, D)), _wspec((1, D)),
              _wspec((D, D)), _wspec((D, D)), _wspec((1, D)),
              _wspec((D, D)), _wspec((1, D)),
              _wspec((D, D)), _wspec((D, D))],
    out_specs=[_nblock(D), _nblock(D), _nblock(D)],
    out_shape=[jax.ShapeDtypeStruct((T, N, D), F32)] * 3,
)

_k_post = pl.pallas_call(
    _k_post_body,
    grid=(T, NB),
    in_specs=[_nblock(D), _nblock(D), _nblock(16),
              _wspec((D, D)), _wspec((1, D)),
              _wspec((D, D)), _wspec((D, D)), _wspec((1, D)),
              _wspec((D, D)), _wspec((1, D)),
              _wspec((D, 2 * G)), _wspec((1, 2 * G)),
              pl.BlockSpec((1, 1, BN), lambda t, i: (t * NB + i, 0, 0)),
              _wspec((G, G)), _wspec((1, G))],
    out_specs=pl.BlockSpec((1, NG, G), lambda t, i: (t, 0, 0)),
    out_shape=jax.ShapeDtypeStruct((T, NG, G), F32),
    scratch_shapes=[pltpu.VMEM((NG, G), F32)],
)


_SC_MESH = plsc.VectorSubcoreMesh(core_axis_name="c", subcore_axis_name="s")


@functools.partial(
    pl.kernel,
    out_type=jax.ShapeDtypeStruct((T * N, D), F32),
    mesh=_SC_MESH,
    scratch_types=[
        pltpu.VMEM_SHARED((N, D), F32),
        pltpu.VMEM((2 * SBK,), jnp.int32),
        pltpu.VMEM((SBK,), jnp.int32),
        pltpu.VMEM((SBK,), jnp.int32),
        pltpu.VMEM((SBK,), jnp.int32),
        pltpu.VMEM((SBK,), jnp.int32),
        pltpu.VMEM((SBK, D), F32),
        pltpu.VMEM((SBK, D), F32),
        pltpu.VMEM((SBK, D), F32),
        pltpu.VMEM((SBK, D), F32),
        pltpu.VMEM((SBK, D), F32),
        pltpu.VMEM((SBK, D), F32),
        pltpu.SemaphoreType.DMA,
        pltpu.SemaphoreType.DMA,
        pltpu.SemaphoreType.DMA,
        pltpu.SemaphoreType.DMA,
        pltpu.SemaphoreType.DMA,
        pltpu.SemaphoreType.DMA,
        pltpu.SemaphoreType.DMA,
        pltpu.SemaphoreType.DMA,
    ],
)
def _k_edge(hs_hbm, hd_hbm, ee_hbm, idx_hbm, out_hbm,
            acc, idxb, fibr, tg, tibr0, tibr1,
            ab0, ab1, bb0, bb1, cb0, cb1,
            sa0, sa1, sb0, sb1, sc0, sc1, ss0, ss1):
    c = lax.axis_index("c")
    s = lax.axis_index("s")
    coff = c * N
    AB, BB, CB = [ab0, ab1], [bb0, bb1], [cb0, cb1]
    TIBR = [tibr0, tibr1]
    SA, SBm, SC, SS = [sa0, sa1], [sb0, sb1], [sc0, sc1], [ss0, ss1]

    def zrow(r, carry):
        for k in range(D // 16):
            cb0[r, pl.ds(k * 16, 16)] = jnp.zeros((16,), F32)
        return carry

    lax.fori_loop(0, CHZ, zrow, 0)
    ch0 = 15 * s + jnp.minimum(s, 10)
    nch = 15 + (s < 10).astype(jnp.int32)

    def zchunk(j, carry):
        pltpu.sync_copy(cb0.at[pl.ds(0, CHZ)],
                        acc.at[pl.ds((ch0 + j) * CHZ, CHZ)])
        return carry

    lax.fori_loop(0, nch, zchunk, 0)
    plsc.subcore_barrier()

    blk0 = NMAIN * s + jnp.minimum(s, 8)

    def prep(q, gblk):
        pltpu.sync_copy(idx_hbm.at[pl.ds((c * GB + gblk) * 2 * SBK, 2 * SBK)],
                        idxb)
        for k in range(SBK // 16):
            sl = pl.ds(k * 16, 16)
            fibr[sl] = idxb[sl] + coff
            t = idxb[pl.ds(SBK + k * 16, 16)]
            TIBR[q][sl] = t
            tg[sl] = t + coff
        e0 = c * E + gblk * SBK
        pltpu.async_copy(hs_hbm.at[fibr], AB[q], SA[q])
        pltpu.async_copy(hd_hbm.at[tg], BB[q], SBm[q])
        pltpu.async_copy(ee_hbm.at[pl.ds(e0, SBK)], CB[q], SC[q])

    def wait_in(q):
        pltpu.make_async_copy(hs_hbm.at[fibr], AB[q], SA[q]).wait()
        pltpu.make_async_copy(hd_hbm.at[tg], BB[q], SBm[q]).wait()
        pltpu.make_async_copy(ee_hbm.at[pl.ds(0, SBK)], CB[q], SC[q]).wait()

    def compute(q):
        def row2(r, rc):
            for rr in range(2):
                ri = 2 * r + rr
                for k in range(D // 16):
                    sl = pl.ds(k * 16, 16)
                    v = AB[q][ri, sl] + BB[q][ri, sl] + CB[q][ri, sl]
                    AB[q][ri, sl] = jnp.maximum(v, 0.0)
            return rc

        lax.fori_loop(0, SBK // 2, row2, 0)

    def scat(q):
        pltpu.async_copy(AB[q], acc.at[TIBR[q]], SS[q], add=True)

    def wait_scat(q):
        pltpu.make_async_copy(AB[q], acc.at[TIBR[q]], SS[q]).wait()

    prep(0, blk0)

    def pair(i, carry):
        b0 = blk0 + 2 * i
        wait_in(0)

        @pl.when(i > 0)
        def _():
            wait_scat(1)

        prep(1, b0 + 1)
        compute(0)
        scat(0)

        wait_in(1)

        @pl.when((i < NPAIR - 1) | (s < 8))
        def _():
            wait_scat(0)
            prep(0, b0 + 2)

        compute(1)
        scat(1)
        return carry

    lax.fori_loop(0, NPAIR, pair, 0)

    @pl.when(s < 8)
    def _():
        wait_in(0)
        wait_scat(1)
        compute(0)
        scat(0)
        wait_scat(0)

    @pl.when(s >= 8)
    def _():
        wait_scat(0)
        wait_scat(1)

    plsc.subcore_barrier()

    def outchunk(j, carry):
        pltpu.sync_copy(acc.at[pl.ds((ch0 + j) * CHZ, CHZ)],
                        out_hbm.at[pl.ds(c * N + (ch0 + j) * CHZ, CHZ)])
        return carry

    lax.fori_loop(0, nch, outchunk, 0)


@functools.partial(
    pl.kernel,
    out_type=jax.ShapeDtypeStruct((T * N, 16), F32),
    mesh=_SC_MESH,
    scratch_types=[
        pltpu.VMEM_SHARED((N, 16), F32),
        pltpu.VMEM((SB,), jnp.int32),
        pltpu.VMEM((SB, 16), F32),
        pltpu.VMEM((CH, 16), F32),
    ],
)
def _k_cnt(ti_hbm, out_hbm, acc, tib, ones, zbuf):
    c = lax.axis_index("c")
    s = lax.axis_index("s")
    ch0 = 7 * s + jnp.minimum(s, 13)
    nch = 7 + (s < 13).astype(jnp.int32)

    def fill(r, carry):
        ones[r, pl.ds(0, 16)] = jnp.full((16,), 1.0, F32)
        return carry

    lax.fori_loop(0, SB, fill, 0)

    def zrow(r, carry):
        zbuf[r, pl.ds(0, 16)] = jnp.zeros((16,), F32)
        return carry

    lax.fori_loop(0, CH, zrow, 0)

    def zchunk(j, carry):
        pltpu.sync_copy(zbuf, acc.at[pl.ds((ch0 + j) * CH, CH)])
        return carry

    lax.fori_loop(0, nch, zchunk, 0)
    plsc.subcore_barrier()

    def blk_body(blk, carry):
        e0 = c * E + s * EPT + blk * SB
        pltpu.sync_copy(ti_hbm.at[pl.ds(e0, SB)], tib)
        pltpu.sync_copy(ones, acc.at[tib], add=True)
        return carry

    lax.fori_loop(0, NBLK, blk_body, 0)
    plsc.subcore_barrier()

    def outchunk(j, carry):
        pltpu.sync_copy(acc.at[pl.ds((ch0 + j) * CH, CH)],
                        out_hbm.at[pl.ds(c * N + (ch0 + j) * CH, CH)])
        return carry

    lax.fori_loop(0, nch, outchunk, 0)


def kernel(x1_node_features, x1_edge_features, x1_from_idx, x1_to_idx,
           x1_graph_idx, x2_node_features, x2_edge_features, x2_from_idx,
           x2_to_idx, x2_graph_idx, n_graphs,
           W_nenc, b_nenc, W_eenc, b_eenc,
           W_msg1, b_msg1, W_msg2, b_msg2,
           W_upd1, b_upd1, W_upd2, b_upd2,
           W_gate, b_gate, W_gout, b_gout):
    del n_graphs
    nf = jnp.stack([x1_node_features, x2_node_features])
    ef = jnp.stack([x1_edge_features, x2_edge_features])
    fi2 = jnp.concatenate([x1_from_idx, x2_from_idx])
    ti2 = jnp.concatenate([x1_to_idx, x2_to_idx])
    idx1d = jnp.concatenate([fi2.reshape(-1, SBK), ti2.reshape(-1, SBK)],
                            axis=1).reshape(-1)
    gi3 = jnp.stack([x1_graph_idx, x2_graph_idx]).reshape(T * NB, 1, BN)

    W1s, W1d, W1e = W_msg1[:D], W_msg1[D:2 * D], W_msg1[2 * D:]
    Wuh, Wua = W_upd1[:D], W_upd1[D:]
    r1 = lambda b: b.reshape(1, -1)

    h, hs, hd = _k_pre(nf, W_nenc, r1(b_nenc), W1s, W1d)
    ee = _k_ee(ef, W_eenc, r1(b_eenc), W1e, r1(b_msg1))
    cnt = _k_cnt(ti2).reshape(T, N, 16)
    ee2 = ee.reshape(T * E, D)

    upd_args = (W_msg2, r1(b_msg2), Wuh, Wua, r1(b_upd1),
                W_upd2, r1(b_upd2))
    s = None
    for layer in range(NPROP):
        s = _k_edge(hs.reshape(T * N, D), hd.reshape(T * N, D),
                    ee2, idx1d).reshape(T, N, D)
        if layer < NPROP - 1:
            h, hs, hd = _k_mid(h, s, cnt, *upd_args, W1s, W1d)

    out = _k_post(h, s, cnt, *upd_args,
                  W_gate, r1(b_gate), gi3, W_gout, r1(b_gout))
    return out[0], out[1]


# speedup vs baseline: 5.1416x; 1.1738x over previous
import functools

import jax
import jax.numpy as jnp
from jax import lax
from jax.experimental import pallas as pl
from jax.experimental.pallas import tpu as pltpu
from jax.experimental.pallas import tpu_sc as plsc

N = 10000
E = 320000
D = 128
DE = 16
G = 128
NG = 128
NPROP = 5
T = 2

BN = 1000
NB = N // BN
EBLK = 4000
NEB = E // EBLK

NSUB = 16
SB = 80
EPT = E // NSUB
NBLK = EPT // SB
CH = 80
NCH = N // CH
SBK = 64
GB = E // SBK
NMAIN = 312
NPAIR = NMAIN // 2
CHZ = 40

F32 = jnp.float32


BF16 = jnp.bfloat16


def _dot(a, b):
    return jnp.dot(a.astype(BF16), b.astype(BF16),
                   preferred_element_type=F32)


def _dot_exact(a, b):
    return jnp.dot(a, b, preferred_element_type=F32,
                   precision=lax.Precision.HIGHEST)


def _k_pre_body(nf, wn, bn, w1s, w1d, h, hs, hd):
    x = _dot(nf[0], wn[...]) + bn[...]
    h[0] = x
    hs[0] = _dot(x, w1s[...])
    hd[0] = _dot(x, w1d[...])


def _k_ee_body(ef, we, be, w1e, bm1, ee):
    t = _dot(ef[0], we[...]) + be[...]
    ee[0] = _dot(t, w1e[...]) + bm1[...]


def _update(h, s, cnt, wm2, bm2, wu1, bu1, wu2, bu2):
    wm2r = wm2[...].astype(BF16).astype(F32)
    agg = _dot_exact(s[0], wm2r) + cnt[0][:, 0:1] * bm2[...]
    u = _dot(jnp.concatenate([h[0], agg], axis=-1), wu1[...]) + bu1[...]
    return h[0] + _dot(jnp.maximum(u, 0.0), wu2[...]) + bu2[...]


def _k_mid_body(h, s, cnt, wm2, bm2, wu1, bu1, wu2, bu2, w1s, w1d,
                hn, hs, hd):
    x = _update(h, s, cnt, wm2, bm2, wu1, bu1, wu2, bu2)
    hn[0] = x
    hs[0] = _dot(x, w1s[...])
    hd[0] = _dot(x, w1d[...])


def _k_post_body(h, s, cnt, wm2, bm2, wu1, bu1, wu2, bu2,
                 wg, bg, gi, wgo, bgo, out, acc):
    i = pl.program_id(1)
    x = _update(h, s, cnt, wm2, bm2, wu1, bu1, wu2, bu2)
    gv = _dot(x, wg[...]) + bg[...]
    gated = jax.nn.sigmoid(gv[:, :G]) * gv[:, G:]
    seg = gi[0, 0]
    onehot = (seg[:, None] ==
              lax.broadcasted_iota(jnp.int32, (BN, NG), 1)).astype(F32)
    contrib = lax.dot_general(onehot, gated, (((0,), (0,)), ((), ())),
                              preferred_element_type=F32,
                              precision=lax.Precision.HIGHEST)

    @pl.when(i == 0)
    def _():
        acc[...] = contrib

    @pl.when(i > 0)
    def _():
        acc[...] = acc[...] + contrib

    @pl.when(i == NB - 1)
    def _():
        out[0] = _dot(acc[...], wgo[...]) + bgo[...]


def _nblock(last):
    return pl.BlockSpec((1, BN, last), lambda t, i: (t, i, 0))


def _wspec(shape):
    return pl.BlockSpec(shape, lambda t, i: tuple(0 for _ in shape))


_k_pre = pl.pallas_call(
    _k_pre_body,
    grid=(T, NB),
    in_specs=[_nblock(D), _wspec((D, D)), _wspec((1, D)),
              _wspec((D, D)), _wspec((D, D))],
    out_specs=[_nblock(D), _nblock(D), _nblock(D)],
    out_shape=[jax.ShapeDtypeStruct((T, N, D), F32)] * 3,
)

_k_ee = pl.pallas_call(
    _k_ee_body,
    grid=(T, NEB),
    in_specs=[pl.BlockSpec((1, EBLK, DE), lambda t, i: (t, i, 0)),
              _wspec((DE, DE)), _wspec((1, DE)),
              _wspec((DE, D)), _wspec((1, D))],
    out_specs=pl.BlockSpec((1, EBLK, D), lambda t, i: (t, i, 0)),
    out_shape=jax.ShapeDtypeStruct((T, E, D), F32),
)

_k_mid = pl.pallas_call(
    _k_mid_body,
    grid=(T, NB),
    in_specs=[_nblock(D), _nblock(D), _nblock(16),
              _wspec((D, D)), _wspec((1, D)),
              _wspec((2 * D, D)), _wspec((1, D)),
              _wspec((D, D)), _wspec((1, D)),
              _wspec((D, D)), _wspec((D, D))],
    out_specs=[_nblock(D), _nblock(D), _nblock(D)],
    out_shape=[jax.ShapeDtypeStruct((T, N, D), F32)] * 3,
)

_k_post = pl.pallas_call(
    _k_post_body,
    grid=(T, NB),
    in_specs=[_nblock(D), _nblock(D), _nblock(16),
              _wspec((D, D)), _wspec((1, D)),
              _wspec((2 * D, D)), _wspec((1, D)),
              _wspec((D, D)), _wspec((1, D)),
              _wspec((D, 2 * G)), _wspec((1, 2 * G)),
              pl.BlockSpec((1, 1, BN), lambda t, i: (t * NB + i, 0, 0)),
              _wspec((G, G)), _wspec((1, G))],
    out_specs=pl.BlockSpec((1, NG, G), lambda t, i: (t, 0, 0)),
    out_shape=jax.ShapeDtypeStruct((T, NG, G), F32),
    scratch_shapes=[pltpu.VMEM((NG, G), F32)],
)


_SC_MESH = plsc.VectorSubcoreMesh(core_axis_name="c", subcore_axis_name="s")


@functools.partial(
    pl.kernel,
    out_type=jax.ShapeDtypeStruct((T * N, D), F32),
    mesh=_SC_MESH,
    scratch_types=[
        pltpu.VMEM_SHARED((N, D), F32),
        pltpu.VMEM((2 * SBK,), jnp.int32),
        pltpu.VMEM((2 * SBK,), jnp.int32),
        pltpu.VMEM((SBK,), jnp.int32),
        pltpu.VMEM((SBK,), jnp.int32),
        pltpu.VMEM((SBK,), jnp.int32),
        pltpu.VMEM((SBK,), jnp.int32),
        pltpu.VMEM((SBK, D), F32),
        pltpu.VMEM((SBK, D), F32),
        pltpu.VMEM((SBK, D), F32),
        pltpu.VMEM((SBK, D), F32),
        pltpu.VMEM((SBK, D), F32),
        pltpu.VMEM((SBK, D), F32),
        pltpu.SemaphoreType.DMA,
        pltpu.SemaphoreType.DMA,
        pltpu.SemaphoreType.DMA,
        pltpu.SemaphoreType.DMA,
        pltpu.SemaphoreType.DMA,
        pltpu.SemaphoreType.DMA,
        pltpu.SemaphoreType.DMA,
        pltpu.SemaphoreType.DMA,
        pltpu.SemaphoreType.DMA,
        pltpu.SemaphoreType.DMA,
    ],
)
def _k_edge(hs_hbm, hd_hbm, ee_hbm, idx_hbm, out_hbm,
            acc, idxb0, idxb1, fibr, tg, tibr0, tibr1,
            ab0, ab1, bb0, bb1, cb0, cb1,
            sa0, sa1, sb0, sb1, sc0, sc1, ss0, ss1, si0, si1):
    c = lax.axis_index("c")
    s = lax.axis_index("s")
    coff = c * N
    AB, BB, CB = [ab0, ab1], [bb0, bb1], [cb0, cb1]
    IDXB, TIBR = [idxb0, idxb1], [tibr0, tibr1]
    SA, SBm, SC, SS = [sa0, sa1], [sb0, sb1], [sc0, sc1], [ss0, ss1]
    SI = [si0, si1]

    def zrow(r, carry):
        for k in range(D // 16):
            cb0[r, pl.ds(k * 16, 16)] = jnp.zeros((16,), F32)
        return carry

    lax.fori_loop(0, CHZ, zrow, 0)
    ch0 = 15 * s + jnp.minimum(s, 10)
    nch = 15 + (s < 10).astype(jnp.int32)

    def zchunk(j, carry):
        pltpu.sync_copy(cb0.at[pl.ds(0, CHZ)],
                        acc.at[pl.ds((ch0 + j) * CHZ, CHZ)])
        return carry

    lax.fori_loop(0, nch, zchunk, 0)
    plsc.subcore_barrier()

    blk0 = NMAIN * s + jnp.minimum(s, 8)

    def idx_start(q, gblk):
        pltpu.async_copy(
            idx_hbm.at[pl.ds((c * GB + gblk) * 2 * SBK, 2 * SBK)],
            IDXB[q], SI[q])

    def gath_start(q, gblk):
        pltpu.make_async_copy(idx_hbm.at[pl.ds(0, 2 * SBK)],
                              IDXB[q], SI[q]).wait()
        for k in range(SBK // 16):
            sl = pl.ds(k * 16, 16)
            fibr[sl] = IDXB[q][sl] + coff
            t = IDXB[q][pl.ds(SBK + k * 16, 16)]
            TIBR[q][sl] = t
            tg[sl] = t + coff
        e0 = c * E + gblk * SBK
        pltpu.async_copy(hs_hbm.at[fibr], AB[q], SA[q])
        pltpu.async_copy(hd_hbm.at[tg], BB[q], SBm[q])
        pltpu.async_copy(ee_hbm.at[pl.ds(e0, SBK)], CB[q], SC[q])

    def wait_in(q):
        pltpu.make_async_copy(hs_hbm.at[fibr], AB[q], SA[q]).wait()
        pltpu.make_async_copy(hd_hbm.at[tg], BB[q], SBm[q]).wait()
        pltpu.make_async_copy(ee_hbm.at[pl.ds(0, SBK)], CB[q], SC[q]).wait()

    def compute(q):
        @plsc.parallel_loop(0, SBK, step=1, unroll=4)
        def _(ri):
            for k in range(D // 16):
                sl = pl.ds(k * 16, 16)
                v = jnp.maximum(
                    AB[q][ri, sl] + BB[q][ri, sl] + CB[q][ri, sl], 0.0)
                AB[q][ri, sl] = v.astype(BF16).astype(F32)

    def scat(q):
        pltpu.async_copy(AB[q], acc.at[TIBR[q]], SS[q], add=True)

    def wait_scat(q):
        pltpu.make_async_copy(AB[q], acc.at[TIBR[q]], SS[q]).wait()

    idx_start(0, blk0)
    gath_start(0, blk0)
    idx_start(1, blk0 + 1)

    def pair(i, carry):
        b0 = blk0 + 2 * i
        more = (i < NPAIR - 1) | (s < 8)

        wait_in(0)

        @pl.when(i > 0)
        def _():
            wait_scat(1)

        gath_start(1, b0 + 1)

        @pl.when(more)
        def _():
            idx_start(0, b0 + 2)

        compute(0)
        scat(0)

        wait_in(1)

        @pl.when(more)
        def _():
            wait_scat(0)
            gath_start(0, b0 + 2)

        @pl.when(i < NPAIR - 1)
        def _():
            idx_start(1, b0 + 3)

        compute(1)
        scat(1)
        return carry

    lax.fori_loop(0, NPAIR, pair, 0)

    @pl.when(s < 8)
    def _():
        wait_in(0)
        wait_scat(1)
        compute(0)
        scat(0)
        wait_scat(0)

    @pl.when(s >= 8)
    def _():
        wait_scat(0)
        wait_scat(1)

    plsc.subcore_barrier()

    def outchunk(j, carry):
        pltpu.sync_copy(acc.at[pl.ds((ch0 + j) * CHZ, CHZ)],
                        out_hbm.at[pl.ds(c * N + (ch0 + j) * CHZ, CHZ)])
        return carry

    lax.fori_loop(0, nch, outchunk, 0)


@functools.partial(
    pl.kernel,
    out_type=jax.ShapeDtypeStruct((T * N, 16), F32),
    mesh=_SC_MESH,
    scratch_types=[
        pltpu.VMEM_SHARED((N, 16), F32),
        pltpu.VMEM((SB,), jnp.int32),
        pltpu.VMEM((SB, 16), F32),
        pltpu.VMEM((CH, 16), F32),
    ],
)
def _k_cnt(ti_hbm, out_hbm, acc, tib, ones, zbuf):
    c = lax.axis_index("c")
    s = lax.axis_index("s")
    ch0 = 7 * s + jnp.minimum(s, 13)
    nch = 7 + (s < 13).astype(jnp.int32)

    def fill(r, carry):
        ones[r, pl.ds(0, 16)] = jnp.full((16,), 1.0, F32)
        return carry

    lax.fori_loop(0, SB, fill, 0)

    def zrow(r, carry):
        zbuf[r, pl.ds(0, 16)] = jnp.zeros((16,), F32)
        return carry

    lax.fori_loop(0, CH, zrow, 0)

    def zchunk(j, carry):
        pltpu.sync_copy(zbuf, acc.at[pl.ds((ch0 + j) * CH, CH)])
        return carry

    lax.fori_loop(0, nch, zchunk, 0)
    plsc.subcore_barrier()

    def blk_body(blk, carry):
        e0 = c * E + s * EPT + blk * SB
        pltpu.sync_copy(ti_hbm.at[pl.ds(e0, SB)], tib)
        pltpu.sync_copy(ones, acc.at[tib], add=True)
        return carry

    lax.fori_loop(0, NBLK, blk_body, 0)
    plsc.subcore_barrier()

    def outchunk(j, carry):
        pltpu.sync_copy(acc.at[pl.ds((ch0 + j) * CH, CH)],
                        out_hbm.at[pl.ds(c * N + (ch0 + j) * CH, CH)])
        return carry

    lax.fori_loop(0, nch, outchunk, 0)


def kernel(x1_node_features, x1_edge_features, x1_from_idx, x1_to_idx,
           x1_graph_idx, x2_node_features, x2_edge_features, x2_from_idx,
           x2_to_idx, x2_graph_idx, n_graphs,
           W_nenc, b_nenc, W_eenc, b_eenc,
           W_msg1, b_msg1, W_msg2, b_msg2,
           W_upd1, b_upd1, W_upd2, b_upd2,
           W_gate, b_gate, W_gout, b_gout):
    del n_graphs
    nf = jnp.stack([x1_node_features, x2_node_features])
    ef = jnp.stack([x1_edge_features, x2_edge_features])
    fi2 = jnp.concatenate([x1_from_idx, x2_from_idx])
    ti2 = jnp.concatenate([x1_to_idx, x2_to_idx])
    idx1d = jnp.concatenate([fi2.reshape(-1, SBK), ti2.reshape(-1, SBK)],
                            axis=1).reshape(-1)
    gi3 = jnp.stack([x1_graph_idx, x2_graph_idx]).reshape(T * NB, 1, BN)

    W1s, W1d, W1e = W_msg1[:D], W_msg1[D:2 * D], W_msg1[2 * D:]
    r1 = lambda b: b.reshape(1, -1)

    h, hs, hd = _k_pre(nf, W_nenc, r1(b_nenc), W1s, W1d)
    ee = _k_ee(ef, W_eenc, r1(b_eenc), W1e, r1(b_msg1))
    cnt = _k_cnt(ti2).reshape(T, N, 16)
    ee2 = ee.reshape(T * E, D)

    upd_args = (W_msg2, r1(b_msg2), W_upd1, r1(b_upd1),
                W_upd2, r1(b_upd2))
    s = None
    for layer in range(NPROP):
        s = _k_edge(hs.reshape(T * N, D), hd.reshape(T * N, D),
                    ee2, idx1d).reshape(T, N, D)
        if layer < NPROP - 1:
            h, hs, hd = _k_mid(h, s, cnt, *upd_args, W1s, W1d)

    out = _k_post(h, s, cnt, *upd_args,
                  W_gate, r1(b_gate), gi3, W_gout, r1(b_gout))
    return out[0], out[1]
